# Initial kernel scaffold; baseline (speedup 1.0000x reference)
import jax
import jax.numpy as jnp
from jax.experimental import pallas as pl


def kernel(itemIDs, emb_table):
    raise NotImplementedError("write your pallas kernel here")


# speedup vs baseline: 1.2982x; 1.2982x over previous
import functools

import jax
import jax.numpy as jnp
from jax import lax
from jax.experimental import pallas as pl
from jax.experimental.pallas import tpu as pltpu
from jax.experimental.pallas import tpu_sc as plsc

TREE_NODE_NUM = 2000000
EMBED_DIM = 64
BATCH = 16384
HIST = 50
TOTAL = BATCH * HIST

_info = plsc.get_sparse_core_info()
_NC = _info.num_cores
_NS = _info.num_subcores
_NW = _NC * _NS

CHUNK = 128
PER_W = TOTAL // _NW
NCHUNK = PER_W // CHUNK

_mesh = plsc.VectorSubcoreMesh(core_axis_name="c", subcore_axis_name="s")


@functools.partial(
    pl.kernel,
    mesh=_mesh,
    out_type=jax.ShapeDtypeStruct((TOTAL, EMBED_DIM), jnp.float32),
    scratch_types=[
        pltpu.VMEM((NCHUNK, CHUNK), jnp.int32),
        pltpu.VMEM((CHUNK, EMBED_DIM), jnp.float32),
        pltpu.SemaphoreType.DMA,
    ],
    compiler_params=pltpu.CompilerParams(use_tc_tiling_on_sc=False),
)
def _sc_gather(idx_hbm, table_hbm, out_hbm, idx_v, rows_v, sem):
    c = lax.axis_index("c")
    s = lax.axis_index("s")
    wid = s * _NC + c
    pltpu.sync_copy(idx_hbm.at[wid], idx_v)
    base = wid * PER_W

    def body(j, carry):
        pltpu.async_copy(table_hbm.at[idx_v.at[j]], rows_v, sem).wait()
        pltpu.sync_copy(rows_v, out_hbm.at[pl.ds(base + j * CHUNK, CHUNK)])
        return carry

    lax.fori_loop(0, NCHUNK, body, 0)


def kernel(itemIDs, emb_table):
    idx = itemIDs.reshape(_NW, NCHUNK, CHUNK).astype(jnp.int32)
    out = _sc_gather(idx, emb_table)
    return out.reshape(BATCH, HIST, EMBED_DIM)


# speedup vs baseline: 1.3911x; 1.0716x over previous
import functools

import jax
import jax.numpy as jnp
from jax import lax
from jax.experimental import pallas as pl
from jax.experimental.pallas import tpu as pltpu
from jax.experimental.pallas import tpu_sc as plsc

TREE_NODE_NUM = 2000000
EMBED_DIM = 64
BATCH = 16384
HIST = 50
TOTAL = BATCH * HIST

_info = plsc.get_sparse_core_info()
_NC = _info.num_cores
_NS = _info.num_subcores
_NW = _NC * _NS

CHUNK = 128
K = 4
GROUP = K * CHUNK
NB = 2
PER_W = TOTAL // _NW
NCHUNK = PER_W // CHUNK
NGROUP = PER_W // GROUP
NITER = NGROUP // NB

_mesh = plsc.VectorSubcoreMesh(core_axis_name="c", subcore_axis_name="s")


@functools.partial(
    pl.kernel,
    mesh=_mesh,
    out_type=jax.ShapeDtypeStruct((TOTAL, EMBED_DIM), jnp.float32),
    scratch_types=[
        pltpu.VMEM((NCHUNK, CHUNK), jnp.int32),
        pltpu.VMEM((NB, GROUP, EMBED_DIM), jnp.float32),
        pltpu.SemaphoreType.DMA,
        pltpu.SemaphoreType.DMA,
        pltpu.SemaphoreType.DMA,
        pltpu.SemaphoreType.DMA,
    ],
    compiler_params=pltpu.CompilerParams(use_tc_tiling_on_sc=False),
)
def _sc_gather(idx_hbm, table_hbm, out_hbm, idx_v, rows_v, g0, g1, s0, s1):
    c = lax.axis_index("c")
    s = lax.axis_index("s")
    wid = s * _NC + c
    gsem = (g0, g1)
    ssem = (s0, s1)
    pltpu.sync_copy(idx_hbm.at[wid], idx_v)
    base = wid * PER_W

    def fire_group(gidx, b):
        for k in range(K):
            pltpu.async_copy(
                table_hbm.at[idx_v.at[gidx * K + k]],
                rows_v.at[b, pl.ds(k * CHUNK, CHUNK)],
                gsem[b],
            )

    def drain_gathers(b):
        pltpu.make_async_copy(
            table_hbm.at[pl.ds(0, GROUP)], rows_v.at[b], gsem[b]
        ).wait()

    def drain_store(b):
        pltpu.make_async_copy(
            rows_v.at[b], out_hbm.at[pl.ds(0, GROUP)], ssem[b]
        ).wait()

    for b in range(NB):
        fire_group(b, b)

    def body(i, carry):
        for b in range(NB):
            g = i * NB + b
            drain_gathers(b)
            pltpu.async_copy(
                rows_v.at[b],
                out_hbm.at[pl.ds(base + g * GROUP, GROUP)],
                ssem[b],
            )
            gn = g + NB

            @pl.when(gn < NGROUP)
            def _refill():
                drain_store(b)
                fire_group(gn, b)

        return carry

    lax.fori_loop(0, NITER, body, 0)
    for b in range(NB):
        drain_store(b)


def kernel(itemIDs, emb_table):
    idx = itemIDs.reshape(_NW, NCHUNK, CHUNK).astype(jnp.int32)
    out = _sc_gather(idx, emb_table)
    return out.reshape(BATCH, HIST, EMBED_DIM)
